# baseline (speedup 1.0000x reference)
import functools

import jax
import jax.numpy as jnp
from jax import lax
from jax.experimental import pallas as pl
from jax.experimental.pallas import tpu as pltpu
from jax.experimental.pallas import tpu_sc as plsc

_B, _C, _T, _H, _W = 8, 3, 32, 224, 224
_NF = 4
_STRIDE = 8
_NBLK = _B * _C * _NF
_NC = 2
_NS = 16
_NW = _NC * _NS
_BLK_PER_W = _NBLK // _NW


def _sc_frame_gather(x3):
    mesh = plsc.VectorSubcoreMesh(core_axis_name="c", subcore_axis_name="s")

    @functools.partial(
        pl.kernel,
        mesh=mesh,
        out_type=jax.ShapeDtypeStruct((_NBLK, _H, _W), jnp.float32),
        scratch_types=[
            pltpu.VMEM((_H, _W), jnp.float32),
        ],
    )
    def k(x_hbm, out_hbm, buf):
        wid = lax.axis_index("s") * _NC + lax.axis_index("c")

        def body(kk, carry):
            g = wid * _BLK_PER_W + kk
            bc = g // _NF
            j = g % _NF
            src = bc * _T + _STRIDE * j
            pltpu.sync_copy(x_hbm.at[src], buf)
            pltpu.sync_copy(buf, out_hbm.at[g])
            return carry

        lax.fori_loop(0, _BLK_PER_W, body, 0)

    return k(x3)


def kernel(x):
    x3 = x.reshape(_B * _C * _T, _H, _W)
    out = _sc_frame_gather(x3)
    return out.reshape(_B, _C, _NF, _H, _W)


# speedup vs baseline: 1.0024x; 1.0024x over previous
import functools

import jax
import jax.numpy as jnp
from jax import lax
from jax.experimental import pallas as pl
from jax.experimental.pallas import tpu as pltpu
from jax.experimental.pallas import tpu_sc as plsc

_B, _C, _T, _H, _W = 8, 3, 32, 224, 224
_NF = 4
_STRIDE = 8
_NBLK = _B * _C * _NF
_NC = 2
_NS = 16
_NW = _NC * _NS
_BLK_PER_W = _NBLK // _NW


def _sc_frame_gather(x3):
    mesh = plsc.VectorSubcoreMesh(core_axis_name="c", subcore_axis_name="s")

    @functools.partial(
        pl.kernel,
        mesh=mesh,
        out_type=jax.ShapeDtypeStruct((_NBLK, _H, _W), jnp.float32),
        scratch_types=(
            [pltpu.VMEM((_H // 2, _W), jnp.float32) for _ in range(4)]
            + [pltpu.SemaphoreType.DMA for _ in range(8)]
        ),
    )
    def k(x_hbm, out_hbm, *scratch):
        bufs = scratch[:4]
        sis = scratch[4:8]
        sos = scratch[8:12]
        wid = lax.axis_index("s") * _NC + lax.axis_index("c")
        nch = 2 * _BLK_PER_W
        hh = _H // 2

        def offs(i):
            g = wid * _BLK_PER_W + i // 2
            half = i % 2
            bc = g // _NF
            j = g % _NF
            src = bc * _T + _STRIDE * j
            return src, g, half * hh

        gathers = [None] * 4
        scatters = [None] * 4
        for i in range(nch):
            s = i % 4
            src, _, row = offs(i)
            if scatters[s] is not None:
                scatters[s].wait()
            gathers[s] = pltpu.async_copy(
                x_hbm.at[src, pl.ds(row, hh)], bufs[s], sis[s]
            )
            if i >= 1:
                p = (i - 1) % 4
                gathers[p].wait()
                _, pg, prow = offs(i - 1)
                scatters[p] = pltpu.async_copy(
                    bufs[p], out_hbm.at[pg, pl.ds(prow, hh)], sos[p]
                )
        p = (nch - 1) % 4
        gathers[p].wait()
        _, lg, lrow = offs(nch - 1)
        scatters[p] = pltpu.async_copy(
            bufs[p], out_hbm.at[lg, pl.ds(lrow, hh)], sos[p]
        )
        for s in range(4):
            if scatters[s] is not None:
                scatters[s].wait()

    return k(x3)


def kernel(x):
    x3 = x.reshape(_B * _C * _T, _H, _W)
    out = _sc_frame_gather(x3)
    return out.reshape(_B, _C, _NF, _H, _W)
